# baseline (speedup 1.0000x reference)
import functools

import jax
import jax.numpy as jnp
from jax import lax
from jax.experimental import pallas as pl
from jax.experimental.pallas import tpu as pltpu
from jax.experimental.pallas import tpu_sc as plsc

NUM_FIELDS = 26
VOCAB_P1 = 100001
EMB_DIM = 32
BATCH = 16384

NC = 2
NS = 16
NW = NC * NS
B_PER_W = BATCH // NW
BB = 8
NCHUNKW = B_PER_W // BB


def _body(x_hbm, tf_hbm, out_hbm, xv, rows, semx0, semx1, semg0, semg1,
          semo0, semo1):
    wid = lax.axis_index("s") * NC + lax.axis_index("c")
    b0w = wid * B_PER_W

    def idx_load(c, p, sem):
        pltpu.async_copy(x_hbm.at[pl.ds(b0w + c * BB, BB)], xv.at[p], sem)

    def idx_wait(p, sem):
        pltpu.make_async_copy(x_hbm.at[pl.ds(0, BB)], xv.at[p], sem).wait()

    def fetch(c, p, sem):
        for b in range(BB):
            v0 = xv[p, b, pl.ds(0, 16)]
            v1 = xv[p, b, pl.ds(NUM_FIELDS - 16, 16)]
            for f in range(NUM_FIELDS):
                r = v0[f] if f < 16 else v1[f - (NUM_FIELDS - 16)]
                pltpu.async_copy(tf_hbm.at[f, r], rows.at[p, b, f], sem)

    def drain(sem):
        pltpu.make_async_copy(out_hbm.at[pl.ds(0, BB)], rows.at[0], sem).wait()

    def store(c, p, sem):
        pltpu.async_copy(rows.at[p], out_hbm.at[pl.ds(b0w + c * BB, BB)], sem)

    idx_load(0, 0, semx0)
    idx_load(1, 1, semx1)
    idx_wait(0, semx0)
    fetch(0, 0, semg0)
    idx_wait(1, semx1)
    fetch(1, 1, semg1)

    def pair_body(k, carry):
        a = 2 * k
        drain(semg0)
        store(a, 0, semo0)

        @pl.when(a + 2 < NCHUNKW)
        def _():
            idx_load(a + 2, 0, semx0)

        drain(semg1)
        store(a + 1, 1, semo1)

        @pl.when(a + 3 < NCHUNKW)
        def _():
            idx_load(a + 3, 1, semx1)

        @pl.when(a + 2 < NCHUNKW)
        def _():
            drain(semo0)
            idx_wait(0, semx0)
            fetch(a + 2, 0, semg0)

        @pl.when(a + 3 < NCHUNKW)
        def _():
            drain(semo1)
            idx_wait(1, semx1)
            fetch(a + 3, 1, semg1)

        return carry

    lax.fori_loop(0, NCHUNKW // 2, pair_body, 0)
    drain(semo0)
    drain(semo1)


_mesh = plsc.VectorSubcoreMesh(core_axis_name="c", subcore_axis_name="s")

_gather = functools.partial(
    pl.kernel,
    mesh=_mesh,
    out_type=jax.ShapeDtypeStruct((BATCH, NUM_FIELDS, EMB_DIM), jnp.float32),
    scratch_types=[
        pltpu.VMEM((2, BB, NUM_FIELDS), jnp.int32),
        pltpu.VMEM((2, BB, NUM_FIELDS, EMB_DIM), jnp.float32),
        pltpu.SemaphoreType.DMA,
        pltpu.SemaphoreType.DMA,
        pltpu.SemaphoreType.DMA,
        pltpu.SemaphoreType.DMA,
        pltpu.SemaphoreType.DMA,
        pltpu.SemaphoreType.DMA,
    ],
    compiler_params=pltpu.CompilerParams(use_tc_tiling_on_sc=True),
)(_body)


def kernel(x, tables):
    return _gather(x, tables)


# speedup vs baseline: 1.0072x; 1.0072x over previous
import functools

import jax
import jax.numpy as jnp
from jax import lax
from jax.experimental import pallas as pl
from jax.experimental.pallas import tpu as pltpu
from jax.experimental.pallas import tpu_sc as plsc

NUM_FIELDS = 26
VOCAB_P1 = 100001
EMB_DIM = 32
BATCH = 16384

NC = 2
NS = 16
NW = NC * NS
B_PER_W = BATCH // NW
BB = 8
NCHUNKW = B_PER_W // BB


def _body(x_hbm, tf_hbm, out_hbm, xv, rows, semx0, semx1, semg0, semg1,
          semo0, semo1):
    wid = lax.axis_index("s") * NC + lax.axis_index("c")
    b0w = wid * B_PER_W

    def idx_load(c, p, sem):
        pltpu.async_copy(x_hbm.at[pl.ds(b0w + c * BB, BB)], xv.at[p], sem)

    def idx_wait(p, sem):
        pltpu.make_async_copy(x_hbm.at[pl.ds(0, BB)], xv.at[p], sem).wait()

    def fetch(c, p, sem):
        def row_body(b, carry):
            v0 = xv[p, b, pl.ds(0, 16)]
            v1 = xv[p, b, pl.ds(NUM_FIELDS - 16, 16)]
            for f in range(NUM_FIELDS):
                r = v0[f] if f < 16 else v1[f - (NUM_FIELDS - 16)]
                pltpu.async_copy(tf_hbm.at[f, r], rows.at[p, b, f], sem)
            return carry

        lax.fori_loop(0, BB, row_body, 0)

    def drain(sem):
        pltpu.make_async_copy(out_hbm.at[pl.ds(0, BB)], rows.at[0], sem).wait()

    def store(c, p, sem):
        pltpu.async_copy(rows.at[p], out_hbm.at[pl.ds(b0w + c * BB, BB)], sem)

    idx_load(0, 0, semx0)
    idx_load(1, 1, semx1)
    idx_wait(0, semx0)
    fetch(0, 0, semg0)
    idx_wait(1, semx1)
    fetch(1, 1, semg1)

    def pair_body(k, carry):
        a = 2 * k
        drain(semg0)
        store(a, 0, semo0)

        @pl.when(a + 2 < NCHUNKW)
        def _():
            idx_load(a + 2, 0, semx0)

        drain(semg1)
        store(a + 1, 1, semo1)

        @pl.when(a + 3 < NCHUNKW)
        def _():
            idx_load(a + 3, 1, semx1)

        @pl.when(a + 2 < NCHUNKW)
        def _():
            drain(semo0)
            idx_wait(0, semx0)
            fetch(a + 2, 0, semg0)

        @pl.when(a + 3 < NCHUNKW)
        def _():
            drain(semo1)
            idx_wait(1, semx1)
            fetch(a + 3, 1, semg1)

        return carry

    lax.fori_loop(0, NCHUNKW // 2, pair_body, 0)
    drain(semo0)
    drain(semo1)


_mesh = plsc.VectorSubcoreMesh(core_axis_name="c", subcore_axis_name="s")

_gather = functools.partial(
    pl.kernel,
    mesh=_mesh,
    out_type=jax.ShapeDtypeStruct((BATCH, NUM_FIELDS, EMB_DIM), jnp.float32),
    scratch_types=[
        pltpu.VMEM((2, BB, NUM_FIELDS), jnp.int32),
        pltpu.VMEM((2, BB, NUM_FIELDS, EMB_DIM), jnp.float32),
        pltpu.SemaphoreType.DMA,
        pltpu.SemaphoreType.DMA,
        pltpu.SemaphoreType.DMA,
        pltpu.SemaphoreType.DMA,
        pltpu.SemaphoreType.DMA,
        pltpu.SemaphoreType.DMA,
    ],
    compiler_params=pltpu.CompilerParams(use_tc_tiling_on_sc=True),
)(_body)


def kernel(x, tables):
    return _gather(x, tables)
